# Initial kernel scaffold; baseline (speedup 1.0000x reference)
import jax
import jax.numpy as jnp
from jax.experimental import pallas as pl


def kernel(users_emb, items_emb, ignore_users, ignore_items):
    raise NotImplementedError("write your pallas kernel here")


# speedup vs baseline: 3.5206x; 3.5206x over previous
import functools

import jax
import jax.numpy as jnp
from jax import lax
from jax.experimental import pallas as pl
from jax.experimental.pallas import tpu as pltpu
from jax.experimental.pallas import tpu_sc as plsc

NU = 1024
DIM = 64
M = 100000
CHW = 128
NCH = 784
MP = NCH * CHW
BLK = 2048
NBLK = MP // BLK
CPB = BLK // CHW
TOPK = 10
NSEL = 16
NEG = -1024.0
NPAIR = 65536
NTILES = 32
PPT = NPAIR // NTILES
PROWS = PPT // CHW
GPT = (NU * NSEL) // NTILES
GROWS = GPT // CHW


def _mm_body(u_ref, it_ref, s_ref):
    j = pl.program_id(0)
    s = lax.dot_general(u_ref[...], it_ref[...], (((1,), (1,)), ((), ())),
                        preferred_element_type=jnp.float32)
    col = j * BLK + lax.broadcasted_iota(jnp.int32, (NU, BLK), 1)
    s_ref[...] = jnp.where(col < M, s, NEG)


def _matmul(users, items_p):
    return pl.pallas_call(
        _mm_body,
        grid=(NBLK,),
        in_specs=[
            pl.BlockSpec((NU, DIM), lambda j: (0, 0)),
            pl.BlockSpec((BLK, DIM), lambda j: (j, 0)),
        ],
        out_specs=pl.BlockSpec((NU, BLK), lambda j: (0, j)),
        out_shape=jax.ShapeDtypeStruct((NU, MP), jnp.float32),
        compiler_params=pltpu.CompilerParams(
            dimension_semantics=("arbitrary",)),
    )(users, items_p)


def _sc_scatter(scores_flat, idx3):
    mesh = plsc.VectorSubcoreMesh(core_axis_name="c", subcore_axis_name="s")

    def stateful(refs):
        s_ref, idx_ref = refs

        @pl.core_map(mesh)
        def _():
            def scoped(idx_v, val_v, sem):
                wid = lax.axis_index("s") * 2 + lax.axis_index("c")
                pltpu.sync_copy(idx_ref.at[wid], idx_v)
                for t in range(CHW // 16):
                    val_v[pl.ds(t * 16, 16)] = jnp.full((16,), NEG,
                                                        jnp.float32)
                cps = [pltpu.make_async_copy(val_v, s_ref.at[idx_v.at[j]],
                                             sem)
                       for j in range(PROWS)]
                for cp in cps:
                    cp.start()
                for cp in cps:
                    cp.wait()

            pl.run_scoped(scoped,
                          pltpu.VMEM((PROWS, CHW), jnp.int32),
                          pltpu.VMEM((CHW,), jnp.float32),
                          pltpu.SemaphoreType.DMA)

    scores_flat, _ = pl.run_state(stateful)((scores_flat, idx3))
    return scores_flat


def _sel_body(s_ref, q_ref, c_ref):
    j = pl.program_id(0)
    s = s_ref[...]
    cm = jnp.max(s.reshape(NU, CPB, CHW), axis=2)
    c_ref[j] = cm

    @pl.when(j == NBLK - 1)
    def _():
        c = jnp.concatenate([c_ref[k] for k in range(NBLK)], axis=1)
        cid = lax.broadcasted_iota(jnp.int32, (NU, NCH), 1)
        big_i = jnp.int32(2 ** 30)
        for k in range(NSEL):
            m = jnp.max(c, axis=1, keepdims=True)
            cand = jnp.where(c == m, cid, big_i)
            g = jnp.min(cand, axis=1, keepdims=True)
            q_ref[:, k:k + 1] = g
            c = jnp.where(cid == g, -jnp.inf, c)


def _select(scores):
    return pl.pallas_call(
        _sel_body,
        grid=(NBLK,),
        in_specs=[pl.BlockSpec((NU, BLK), lambda j: (0, j))],
        out_specs=pl.BlockSpec((NU, NSEL), lambda j: (0, 0)),
        out_shape=jax.ShapeDtypeStruct((NU, NSEL), jnp.int32),
        scratch_shapes=[pltpu.VMEM((NBLK, NU, CPB), jnp.float32)],
        compiler_params=pltpu.CompilerParams(
            dimension_semantics=("arbitrary",),
            vmem_limit_bytes=100 * 1024 * 1024),
    )(scores)


def _sc_gather(cidx3, sview):
    mesh = plsc.VectorSubcoreMesh(core_axis_name="c", subcore_axis_name="s")

    @functools.partial(
        pl.kernel,
        out_type=jax.ShapeDtypeStruct((NU * NSEL, CHW), jnp.float32),
        mesh=mesh,
        scratch_types=[
            pltpu.VMEM((GROWS, CHW), jnp.int32),
            pltpu.VMEM((GPT, CHW), jnp.float32),
            pltpu.SemaphoreType.DMA,
        ],
    )
    def k(cidx_hbm, sview_hbm, out_hbm, idx_v, buf, sem):
        wid = lax.axis_index("s") * 2 + lax.axis_index("c")
        pltpu.sync_copy(cidx_hbm.at[wid], idx_v)
        cps = [pltpu.make_async_copy(sview_hbm.at[idx_v.at[t]],
                                     buf.at[pl.ds(t * CHW, CHW)], sem)
               for t in range(GROWS)]
        for cp in cps:
            cp.start()
        for cp in cps:
            cp.wait()
        pltpu.sync_copy(buf, out_hbm.at[pl.ds(wid * GPT, GPT)])

    return k(cidx3, sview)


def _top_body(v_ref, gi_ref, tv_ref, ti_ref):
    v = v_ref[...]
    gi = gi_ref[...]
    big_i = jnp.int32(2 ** 30)
    for k in range(TOPK):
        m = jnp.max(v, axis=1, keepdims=True)
        cand = jnp.where(v == m, gi, big_i)
        g = jnp.min(cand, axis=1, keepdims=True)
        tv_ref[:, k:k + 1] = m
        ti_ref[:, k:k + 1] = g
        v = jnp.where(gi == g, -jnp.inf, v)


def _topk(vals, gidx):
    return pl.pallas_call(
        _top_body,
        out_shape=(jax.ShapeDtypeStruct((NU, TOPK), jnp.float32),
                   jax.ShapeDtypeStruct((NU, TOPK), jnp.int32)),
    )(vals, gidx)


def kernel(users_emb, items_emb, ignore_users, ignore_items):
    items_p = jnp.pad(items_emb, ((0, MP - M), (0, 0)))
    scores = _matmul(users_emb, items_p)

    u = ignore_users.astype(jnp.int32)
    i = ignore_items.astype(jnp.int32)
    flat = (u * MP + i).reshape(NTILES, PROWS, CHW)
    scores_flat = _sc_scatter(scores.reshape(NU * MP), flat)

    qid = _select(scores_flat.reshape(NU, MP))

    cidx = (jnp.arange(NU, dtype=jnp.int32)[:, None] * NCH + qid)
    gathered = _sc_gather(cidx.reshape(NTILES, GROWS, CHW),
                          scores_flat.reshape(NU * NCH, CHW))

    pool = gathered.reshape(NU, NSEL * CHW)
    lane = jnp.arange(NSEL * CHW, dtype=jnp.int32) % CHW
    gi = jnp.repeat(qid, CHW, axis=1) * CHW + lane[None, :]
    top_vals, top_items = _topk(pool, gi)
    return top_vals, top_items


# speedup vs baseline: 7.3946x; 2.1004x over previous
import functools

import jax
import jax.numpy as jnp
from jax import lax
from jax.experimental import pallas as pl
from jax.experimental.pallas import tpu as pltpu
from jax.experimental.pallas import tpu_sc as plsc

NU = 1024
DIM = 64
M = 100000
CHW = 128
NCH = 784
MP = NCH * CHW
BLK = 2048
NBLK = MP // BLK
CPB = BLK // CHW
TOPK = 10
NSEL = 16
NEG = -1024.0
NPAIR = 65536
NTILES = 32
PPT = NPAIR // NTILES
PROWS = PPT // CHW
GPT = (NU * NSEL) // NTILES
GROWS = GPT // CHW


def _mm_body(u_ref, it_ref, s_ref):
    j = pl.program_id(0)
    s = lax.dot_general(u_ref[...], it_ref[...], (((1,), (1,)), ((), ())),
                        preferred_element_type=jnp.float32)
    col = j * BLK + lax.broadcasted_iota(jnp.int32, (NU, BLK), 1)
    s = jnp.where(col < M, s, NEG)
    s_ref[...] = s.reshape(NU, CPB, CHW)


def _matmul(users, items_p):
    return pl.pallas_call(
        _mm_body,
        grid=(NBLK,),
        in_specs=[
            pl.BlockSpec((NU, DIM), lambda j: (0, 0)),
            pl.BlockSpec((BLK, DIM), lambda j: (j, 0)),
        ],
        out_specs=pl.BlockSpec((NU, CPB, CHW), lambda j: (0, j, 0)),
        out_shape=jax.ShapeDtypeStruct((NU, NCH, CHW), jnp.float32),
        compiler_params=pltpu.CompilerParams(
            dimension_semantics=("arbitrary",)),
    )(users, items_p)


def _sc_scatter(scores_flat, idx3):
    mesh = plsc.VectorSubcoreMesh(core_axis_name="c", subcore_axis_name="s")

    def stateful(refs):
        s_ref, idx_ref = refs

        @pl.core_map(mesh)
        def _():
            def scoped(idx_v, val_v, sem):
                wid = lax.axis_index("s") * 2 + lax.axis_index("c")
                pltpu.sync_copy(idx_ref.at[wid], idx_v)
                for t in range(CHW // 16):
                    val_v[pl.ds(t * 16, 16)] = jnp.full((16,), NEG,
                                                        jnp.float32)
                cps = [pltpu.make_async_copy(val_v, s_ref.at[idx_v.at[j]],
                                             sem)
                       for j in range(PROWS)]
                for cp in cps:
                    cp.start()
                for cp in cps:
                    cp.wait()

            pl.run_scoped(scoped,
                          pltpu.VMEM((PROWS, CHW), jnp.int32),
                          pltpu.VMEM((CHW,), jnp.float32),
                          pltpu.SemaphoreType.DMA)

    scores_flat, _ = pl.run_state(stateful)((scores_flat, idx3))
    return scores_flat


def _sel_body(s_ref, q_ref, c_ref):
    j = pl.program_id(0)
    cm = jnp.max(s_ref[...], axis=2)
    c_ref[j] = cm

    @pl.when(j == NBLK - 1)
    def _():
        c = jnp.concatenate([c_ref[k] for k in range(NBLK)], axis=1)
        cid = lax.broadcasted_iota(jnp.int32, (NU, NCH), 1)
        big_i = jnp.int32(2 ** 30)
        for k in range(NSEL):
            m = jnp.max(c, axis=1, keepdims=True)
            cand = jnp.where(c == m, cid, big_i)
            g = jnp.min(cand, axis=1, keepdims=True)
            q_ref[:, k:k + 1] = g
            c = jnp.where(cid == g, -jnp.inf, c)


def _select(scores3):
    return pl.pallas_call(
        _sel_body,
        grid=(NBLK,),
        in_specs=[pl.BlockSpec((NU, CPB, CHW), lambda j: (0, j, 0))],
        out_specs=pl.BlockSpec((NU, NSEL), lambda j: (0, 0)),
        out_shape=jax.ShapeDtypeStruct((NU, NSEL), jnp.int32),
        scratch_shapes=[pltpu.VMEM((NBLK, NU, CPB), jnp.float32)],
        compiler_params=pltpu.CompilerParams(
            dimension_semantics=("arbitrary",),
            vmem_limit_bytes=100 * 1024 * 1024),
    )(scores3)


def _sc_gather(cidx3, sview):
    mesh = plsc.VectorSubcoreMesh(core_axis_name="c", subcore_axis_name="s")

    @functools.partial(
        pl.kernel,
        out_type=jax.ShapeDtypeStruct((NU * NSEL, CHW), jnp.float32),
        mesh=mesh,
        scratch_types=[
            pltpu.VMEM((GROWS, CHW), jnp.int32),
            pltpu.VMEM((GPT, CHW), jnp.float32),
            pltpu.SemaphoreType.DMA,
        ],
    )
    def k(cidx_hbm, sview_hbm, out_hbm, idx_v, buf, sem):
        wid = lax.axis_index("s") * 2 + lax.axis_index("c")
        pltpu.sync_copy(cidx_hbm.at[wid], idx_v)
        cps = [pltpu.make_async_copy(sview_hbm.at[idx_v.at[t]],
                                     buf.at[pl.ds(t * CHW, CHW)], sem)
               for t in range(GROWS)]
        for cp in cps:
            cp.start()
        for cp in cps:
            cp.wait()
        pltpu.sync_copy(buf, out_hbm.at[pl.ds(wid * GPT, GPT)])

    return k(cidx3, sview)


def _top_body(v_ref, gi_ref, tv_ref, ti_ref):
    v = v_ref[...].reshape(NU, NSEL * CHW)
    gi = gi_ref[...].reshape(NU, NSEL * CHW)
    big_i = jnp.int32(2 ** 30)
    for k in range(TOPK):
        m = jnp.max(v, axis=1, keepdims=True)
        cand = jnp.where(v == m, gi, big_i)
        g = jnp.min(cand, axis=1, keepdims=True)
        tv_ref[:, k:k + 1] = m
        ti_ref[:, k:k + 1] = g
        v = jnp.where(gi == g, -jnp.inf, v)


def _topk(vals, gidx):
    return pl.pallas_call(
        _top_body,
        out_shape=(jax.ShapeDtypeStruct((NU, TOPK), jnp.float32),
                   jax.ShapeDtypeStruct((NU, TOPK), jnp.int32)),
    )(vals, gidx)


def kernel(users_emb, items_emb, ignore_users, ignore_items):
    items_p = jnp.pad(items_emb, ((0, MP - M), (0, 0)))
    scores3 = _matmul(users_emb, items_p)

    u = ignore_users.astype(jnp.int32)
    i = ignore_items.astype(jnp.int32)
    flat = (u * MP + i).reshape(NTILES, PROWS, CHW)
    scores_flat = _sc_scatter(scores3.reshape(NU * MP), flat)

    qid = _select(scores_flat.reshape(NU, NCH, CHW))

    cidx = (jnp.arange(NU, dtype=jnp.int32)[:, None] * NCH + qid)
    gathered = _sc_gather(cidx.reshape(NTILES, GROWS, CHW),
                          scores_flat.reshape(NU * NCH, CHW))

    pool3 = gathered.reshape(NU, NSEL, CHW)
    gi3 = (qid[:, :, None] * CHW
           + jnp.arange(CHW, dtype=jnp.int32)[None, None, :])
    top_vals, top_items = _topk(pool3, gi3)
    return top_vals, top_items
